# baseline (device time: 16108 ns/iter reference)
import jax
import jax.numpy as jnp
from jax import lax
from jax.experimental import pallas as pl
from jax.experimental.pallas import tpu as pltpu

N_DEV = 8


def kernel(x, pi):
    _, m, n = x.shape

    def body(x_hbm, pi_hbm, out_ref, x_vmem, comm_ref, pi_s,
             pi_sem, x_sem, send_sem, recv_sem):
        cp_pi = pltpu.make_async_copy(pi_hbm, pi_s, pi_sem)
        cp_pi.start()
        cp_x = pltpu.make_async_copy(x_hbm, x_vmem, x_sem)
        cp_x.start()
        cp_pi.wait()

        my = lax.axis_index("i")
        dst = pi_s[my]
        src = jnp.int32(0)
        for j in range(N_DEV):
            src = jnp.where(pi_s[j] == my, jnp.int32(j), src)

        barrier = pltpu.get_barrier_semaphore()
        pl.semaphore_signal(barrier, inc=1, device_id=(dst,),
                            device_id_type=pl.DeviceIdType.MESH)
        pl.semaphore_signal(barrier, inc=1, device_id=(src,),
                            device_id_type=pl.DeviceIdType.MESH)

        cp_x.wait()
        comm_ref[0] = x_vmem[0].astype(jnp.bfloat16)

        pl.semaphore_wait(barrier, 2)

        rdma = pltpu.make_async_remote_copy(
            src_ref=comm_ref,
            dst_ref=out_ref,
            send_sem=send_sem,
            recv_sem=recv_sem,
            device_id=(dst,),
            device_id_type=pl.DeviceIdType.MESH,
        )
        rdma.start()
        rdma.wait()

        def exit_barrier(sem):
            pl.semaphore_signal(sem, inc=1, device_id=(dst,),
                                device_id_type=pl.DeviceIdType.MESH)
            pl.semaphore_signal(sem, inc=1, device_id=(src,),
                                device_id_type=pl.DeviceIdType.MESH)
            pl.semaphore_wait(sem, 2)

        pl.run_scoped(exit_barrier, pltpu.SemaphoreType.REGULAR)

    return pl.pallas_call(
        body,
        out_shape=jax.ShapeDtypeStruct((1, m, n), jnp.bfloat16),
        in_specs=[
            pl.BlockSpec(memory_space=pl.ANY),
            pl.BlockSpec(memory_space=pl.ANY),
        ],
        out_specs=pl.BlockSpec(memory_space=pltpu.MemorySpace.HBM),
        scratch_shapes=[
            pltpu.VMEM((1, m, n), jnp.float32),
            pltpu.VMEM((1, m, n), jnp.bfloat16),
            pltpu.SMEM((N_DEV,), jnp.int32),
            pltpu.SemaphoreType.DMA,
            pltpu.SemaphoreType.DMA,
            pltpu.SemaphoreType.DMA,
            pltpu.SemaphoreType.DMA,
        ],
        compiler_params=pltpu.CompilerParams(collective_id=0),
    )(x, pi)


# device time: 13067 ns/iter; 1.2327x vs baseline; 1.2327x over previous
import jax
import jax.numpy as jnp
from jax import lax
from jax.experimental import pallas as pl
from jax.experimental.pallas import tpu as pltpu

N_DEV = 8


def kernel(x, pi):
    _, m, n = x.shape

    def body(x_ref, pi_ref, out_ref, comm_ref, send_sem, recv_sem):
        my = lax.axis_index("i")
        dst = pi_ref[my]
        src = jnp.int32(0)
        for j in range(N_DEV):
            src = jnp.where(pi_ref[j] == my, jnp.int32(j), src)

        barrier = pltpu.get_barrier_semaphore()
        pl.semaphore_signal(barrier, inc=1, device_id=(src,),
                            device_id_type=pl.DeviceIdType.MESH)

        comm_ref[0] = x_ref[0].astype(jnp.bfloat16)

        pl.semaphore_wait(barrier, 1)

        rdma = pltpu.make_async_remote_copy(
            src_ref=comm_ref,
            dst_ref=out_ref,
            send_sem=send_sem,
            recv_sem=recv_sem,
            device_id=(dst,),
            device_id_type=pl.DeviceIdType.MESH,
        )
        rdma.start()
        rdma.wait()

    return pl.pallas_call(
        body,
        out_shape=jax.ShapeDtypeStruct((1, m, n), jnp.bfloat16),
        in_specs=[
            pl.BlockSpec(memory_space=pltpu.VMEM),
            pl.BlockSpec(memory_space=pltpu.SMEM),
        ],
        out_specs=pl.BlockSpec(memory_space=pltpu.VMEM),
        scratch_shapes=[
            pltpu.VMEM((1, m, n), jnp.bfloat16),
            pltpu.SemaphoreType.DMA,
            pltpu.SemaphoreType.DMA,
        ],
        compiler_params=pltpu.CompilerParams(collective_id=0),
    )(x, pi)
